# baseline (speedup 1.0000x reference)
import functools

import jax
import jax.numpy as jnp
from jax import lax
from jax.experimental import pallas as pl
from jax.experimental.pallas import tpu as pltpu
from jax.experimental.pallas import tpu_sc as plsc

B, T = 16, 200
H_PIX, W_PIX = 27, 48
PPF = H_PIX * W_PIX
WPF = PPF * 3
FRAMES = B * T
NBINS = 512
LOOKUP = 101
ODIM = 128

NC, NS = 2, 16
NW = NC * NS
FPW = FRAMES // NW
CHUNK = 10
NCH = FPW // CHUNK
GROUPS = PPF // 16


def _sc_hist_body(pix_hbm, hist_hbm, pixbuf0, pixbuf1, histbuf0, histbuf1,
                  sem_in0, sem_in1, sem_out0, sem_out1):
  cid = lax.axis_index("c")
  sid = lax.axis_index("s")
  wid = sid * NC + cid
  word0 = wid * (FPW * WPF)
  elem0 = wid * (FPW * NBINS)

  pixbufs = [pixbuf0, pixbuf1]
  histbufs = [histbuf0, histbuf1]
  sems_in = [sem_in0, sem_in1]
  sems_out = [sem_out0, sem_out1]

  iota = lax.iota(jnp.int32, 16)
  iota3 = iota * 3

  def start_in(c, slot):
    return pltpu.async_copy(
        pix_hbm.at[pl.ds(word0 + c * (CHUNK * WPF), CHUNK * WPF)],
        pixbufs[slot], sems_in[slot])

  def start_out(c, slot):
    return pltpu.async_copy(
        histbufs[slot],
        hist_hbm.at[pl.ds(elem0 + c * (CHUNK * NBINS), CHUNK * NBINS)],
        sems_out[slot])

  in_handles = {0: start_in(0, 0)}
  out_handles = {}

  for c in range(NCH):
    slot = c & 1
    in_handles.pop(c).wait()
    if c + 1 < NCH:
      in_handles[c + 1] = start_in(c + 1, slot ^ 1)
    if c >= 2:
      out_handles.pop(c - 2).wait()

    zeros16 = jnp.zeros((16,), jnp.float32)

    def zero_body(i, _, hb=histbufs[slot]):
      base = i * 128
      for u in range(8):
        hb[pl.ds(base + u * 16, 16)] = zeros16
      return _

    lax.fori_loop(0, (CHUNK * NBINS) // 128, zero_body, 0, unroll=False)

    for f in range(CHUNK):
      pix0 = f * WPF
      bin0 = f * NBINS

      def group_body(g, _, pb=pixbufs[slot], hb=histbufs[slot],
                     pix0=pix0, bin0=bin0):
        base = pix0 + g * 48
        idx = base + iota3
        r = plsc.load_gather(pb, [idx])
        gg = plsc.load_gather(pb, [idx + 1])
        bb = plsc.load_gather(pb, [idx + 2])
        bins = ((r & 0xE0) << 1) | ((gg & 0xE0) >> 2) | (bb >> 5)
        cnt, last = plsc.scan_count(bins)
        plsc.addupdate_scatter(hb, [bins + bin0], cnt.astype(jnp.float32),
                               mask=last)
        return _

      lax.fori_loop(0, GROUPS, group_body, 0, unroll=False)

    out_handles[c] = start_out(c, slot)

  out_handles.pop(NCH - 2).wait()
  out_handles.pop(NCH - 1).wait()


def _sc_histograms(pix_flat):
  mesh = plsc.VectorSubcoreMesh(core_axis_name="c", subcore_axis_name="s")
  kern = pl.kernel(
      _sc_hist_body,
      out_type=jax.ShapeDtypeStruct((FRAMES * NBINS,), jnp.float32),
      mesh=mesh,
      scratch_types=[
          pltpu.VMEM((CHUNK * WPF,), jnp.int32),
          pltpu.VMEM((CHUNK * WPF,), jnp.int32),
          pltpu.VMEM((CHUNK * NBINS,), jnp.float32),
          pltpu.VMEM((CHUNK * NBINS,), jnp.float32),
          pltpu.SemaphoreType.DMA,
          pltpu.SemaphoreType.DMA,
          pltpu.SemaphoreType.DMA,
          pltpu.SemaphoreType.DMA,
      ],
      compiler_params=pltpu.CompilerParams(needs_layout_passes=False),
  )
  return kern(pix_flat)


def _tc_body(hist_ref, wp_ref, bv_ref, out_ref):
  h = hist_ref[0]
  sq = jnp.sum(h * h, axis=1, keepdims=True)
  inv = lax.rsqrt(jnp.maximum(sq, 1e-24))
  x = h * inv
  g = lax.dot_general(x, x, (((1,), (1,)), ((), ())),
                      preferred_element_type=jnp.float32)
  gp = jnp.concatenate(
      [jnp.zeros((T, 50), jnp.float32), g, jnp.zeros((T, 70), jnp.float32)],
      axis=1)
  row8 = lax.broadcasted_iota(jnp.int32, (8, 128), 0)
  blocks = []
  for i in range(T // 8):
    a = gp[8 * i:8 * i + 8, 8 * i:8 * i + 128]
    for bit in (1, 2, 4):
      a = jnp.where((row8 & bit) > 0, jnp.roll(a, -bit, axis=1), a)
    blocks.append(a)
  sims = jnp.concatenate(blocks, axis=0)
  out = jnp.dot(sims, wp_ref[...], preferred_element_type=jnp.float32)
  out_ref[0] = jnp.maximum(out + bv_ref[...], 0.0)


def _tc_similarity(hist, wp, bv):
  return pl.pallas_call(
      _tc_body,
      grid=(B,),
      in_specs=[
          pl.BlockSpec((1, T, NBINS), lambda i: (i, 0, 0)),
          pl.BlockSpec((128, ODIM), lambda i: (0, 0)),
          pl.BlockSpec((1, ODIM), lambda i: (0, 0)),
      ],
      out_specs=pl.BlockSpec((1, T, ODIM), lambda i: (i, 0, 0)),
      out_shape=jax.ShapeDtypeStruct((B, T, ODIM), jnp.float32),
  )(hist, wp, bv)


@jax.jit
def kernel(inputs, W, b):
  pix_flat = inputs.reshape(-1)
  hist = _sc_histograms(pix_flat).reshape(B, T, NBINS)
  wp = jnp.concatenate(
      [W.T.astype(jnp.float32), jnp.zeros((128 - LOOKUP, ODIM), jnp.float32)],
      axis=0)
  bv = b.reshape(1, ODIM).astype(jnp.float32)
  return _tc_similarity(hist, wp, bv)


# speedup vs baseline: 8.6043x; 8.6043x over previous
import jax
import jax.numpy as jnp
from jax import lax
from jax.experimental import pallas as pl
from jax.experimental.pallas import tpu as pltpu
from jax.experimental.pallas import tpu_sc as plsc

B, T = 16, 200
PPF = 27 * 48
FRAMES = B * T
NBINS = 512
LOOKUP = 101
ODIM = 128

NC, NS = 2, 16
NW = NC * NS
FPW = FRAMES // NW
CHUNK = 10
NCH = FPW // CHUNK
GROUPS = PPF // 16


def _sc_hist_body(pix_hbm, hist_hbm, pixbuf0, pixbuf1, histbuf0, histbuf1,
                  sem_in0, sem_in1, sem_out0, sem_out1):
  cid = lax.axis_index("c")
  sid = lax.axis_index("s")
  wid = sid * NC + cid
  word0 = wid * (FPW * PPF)
  elem0 = wid * (FPW * NBINS)

  pixbufs = [pixbuf0, pixbuf1]
  histbufs = [histbuf0, histbuf1]
  sems_in = [sem_in0, sem_in1]
  sems_out = [sem_out0, sem_out1]

  zeros16 = jnp.zeros((16,), jnp.float32)

  def start_in(c, slot):
    return pltpu.async_copy(
        pix_hbm.at[pl.ds(word0 + c * (CHUNK * PPF), CHUNK * PPF)],
        pixbufs[slot], sems_in[slot])

  def start_out(c, slot):
    return pltpu.async_copy(
        histbufs[slot],
        hist_hbm.at[pl.ds(elem0 + c * (CHUNK * NBINS), CHUNK * NBINS)],
        sems_out[slot])

  in_handles = {0: start_in(0, 0)}
  out_handles = {}

  for c in range(NCH):
    slot = c & 1
    in_handles.pop(c).wait()
    if c + 1 < NCH:
      in_handles[c + 1] = start_in(c + 1, slot ^ 1)
    if c >= 2:
      out_handles.pop(c - 2).wait()

    hb = histbufs[slot]
    pb = pixbufs[slot]

    def zero_body(i, carry, hb=hb):
      base = i * 128
      for u in range(8):
        hb[pl.ds(base + u * 16, 16)] = zeros16
      return carry

    lax.fori_loop(0, (CHUNK * NBINS) // 128, zero_body, 0, unroll=False)

    for f in range(CHUNK):
      pix0 = f * PPF
      bin0 = f * NBINS

      def group_body(g, carry, pb=pb, hb=hb, pix0=pix0, bin0=bin0):
        w = pb[pl.ds(pix0 + g * 16, 16)]
        bins = ((w & 0xE0) << 1) | ((w >> 10) & 0x38) | ((w >> 21) & 7)
        cnt, last = plsc.scan_count(bins)
        plsc.addupdate_scatter(hb, [bins + bin0], cnt.astype(jnp.float32),
                               mask=last)
        return carry

      lax.fori_loop(0, GROUPS, group_body, 0, unroll=False)

    out_handles[c] = start_out(c, slot)

  out_handles.pop(NCH - 2).wait()
  out_handles.pop(NCH - 1).wait()


def _sc_histograms(pix_flat):
  mesh = plsc.VectorSubcoreMesh(core_axis_name="c", subcore_axis_name="s")
  kern = pl.kernel(
      _sc_hist_body,
      out_type=jax.ShapeDtypeStruct((FRAMES * NBINS,), jnp.float32),
      mesh=mesh,
      scratch_types=[
          pltpu.VMEM((CHUNK * PPF,), jnp.int32),
          pltpu.VMEM((CHUNK * PPF,), jnp.int32),
          pltpu.VMEM((CHUNK * NBINS,), jnp.float32),
          pltpu.VMEM((CHUNK * NBINS,), jnp.float32),
          pltpu.SemaphoreType.DMA,
          pltpu.SemaphoreType.DMA,
          pltpu.SemaphoreType.DMA,
          pltpu.SemaphoreType.DMA,
      ],
      compiler_params=pltpu.CompilerParams(needs_layout_passes=False),
  )
  return kern(pix_flat)


def _tc_body(hist_ref, wp_ref, bv_ref, out_ref):
  h = hist_ref[0]
  sq = jnp.sum(h * h, axis=1, keepdims=True)
  inv = lax.rsqrt(jnp.maximum(sq, 1e-24))
  x = h * inv
  g = lax.dot_general(x, x, (((1,), (1,)), ((), ())),
                      preferred_element_type=jnp.float32)
  gp = jnp.concatenate(
      [jnp.zeros((T, 50), jnp.float32), g, jnp.zeros((T, 70), jnp.float32)],
      axis=1)
  row8 = lax.broadcasted_iota(jnp.int32, (8, 128), 0)
  blocks = []
  for i in range(T // 8):
    a = gp[8 * i:8 * i + 8, 8 * i:8 * i + 128]
    for bit in (1, 2, 4):
      a = jnp.where((row8 & bit) > 0, jnp.roll(a, -bit, axis=1), a)
    blocks.append(a)
  sims = jnp.concatenate(blocks, axis=0)
  out = jnp.dot(sims, wp_ref[...], preferred_element_type=jnp.float32)
  out_ref[0] = jnp.maximum(out + bv_ref[...], 0.0)


def _tc_similarity(hist, wp, bv):
  return pl.pallas_call(
      _tc_body,
      grid=(B,),
      in_specs=[
          pl.BlockSpec((1, T, NBINS), lambda i: (i, 0, 0)),
          pl.BlockSpec((128, ODIM), lambda i: (0, 0)),
          pl.BlockSpec((1, ODIM), lambda i: (0, 0)),
      ],
      out_specs=pl.BlockSpec((1, T, ODIM), lambda i: (i, 0, 0)),
      out_shape=jax.ShapeDtypeStruct((B, T, ODIM), jnp.float32),
  )(hist, wp, bv)


@jax.jit
def kernel(inputs, W, b):
  fr = inputs.reshape(FRAMES * PPF, 3)
  packed = fr[:, 0] | (fr[:, 1] << 8) | (fr[:, 2] << 16)
  hist = _sc_histograms(packed).reshape(B, T, NBINS)
  wp = jnp.concatenate(
      [W.T.astype(jnp.float32), jnp.zeros((128 - LOOKUP, ODIM), jnp.float32)],
      axis=0)
  bv = b.reshape(1, ODIM).astype(jnp.float32)
  return _tc_similarity(hist, wp, bv)


# speedup vs baseline: 73.4364x; 8.5348x over previous
import jax
import jax.numpy as jnp
from jax import lax
from jax.experimental import pallas as pl
from jax.experimental.pallas import tpu as pltpu
from jax.experimental.pallas import tpu_sc as plsc

B, T = 16, 200
HP, WP = 27, 48
FRAMES = B * T
NBINS = 512
LOOKUP = 101
ODIM = 128

NW = 32
H0 = 14
NCHUNK = H0 * 3
HISTW = T * NBINS


def _sc_hist_body(z_hbm, hist_hbm, buf0, buf1, hist, sem0, sem1, semo):
  cid = lax.axis_index("c")
  sid = lax.axis_index("s")
  wid = sid * 2 + cid
  b = wid >> 1
  hh = wid & 1
  is0 = hh == 0

  bufs = [buf0, buf1]
  sems = [sem0, sem1]

  iota = lax.iota(jnp.int32, 16)
  iota512 = iota * NBINS
  ones16 = jnp.ones((16,), jnp.float32)
  zeros16 = jnp.zeros((16,), jnp.float32)

  def zero_body(i, carry):
    base = i * 128
    for u in range(8):
      hist[pl.ds(base + u * 16, 16)] = zeros16
    return carry

  lax.fori_loop(0, HISTW // 128, zero_body, 0, unroll=False)

  def dma_in(k, slot):
    h = H0 * hh + (k // 3)
    wc = k % 3
    return pltpu.make_async_copy(
        z_hbm.at[b, h, :, pl.ds(16 * wc, 16), :], bufs[slot], sems[slot])

  def compute(slot):
    buf = bufs[slot]

    def body(i, carry):
      w = i >> 4
      u4 = (i & 15) << 4
      t0 = jnp.minimum(u4, T - 16)
      r = buf[0, w, pl.ds(t0, 16)]
      g = buf[1, w, pl.ds(t0, 16)]
      bb = buf[2, w, pl.ds(t0, 16)]
      bins = ((r & 0xE0) << 1) | ((g & 0xE0) >> 2) | (bb >> 5)
      idx = (t0 * NBINS + iota512) + bins
      mask = (t0 + iota) >= u4
      plsc.addupdate_scatter(hist, [idx], ones16, mask=mask)
      return carry

    lax.fori_loop(0, 16 * 16, body, 0, unroll=2)

  dma_in(0, 0).start()
  for k in range(NCHUNK):
    slot = k & 1
    if k < NCHUNK - 3:
      dma_in(k, slot).wait()
      if k + 1 < NCHUNK - 3:
        dma_in(k + 1, slot ^ 1).start()
      else:
        @pl.when(is0)
        def _(k=k, slot=slot):
          dma_in(k + 1, slot ^ 1).start()
      compute(slot)
    else:
      @pl.when(is0)
      def _(k=k, slot=slot):
        dma_in(k, slot).wait()
        if k + 1 < NCHUNK:
          dma_in(k + 1, slot ^ 1).start()
        compute(slot)

  out = pltpu.make_async_copy(
      hist, hist_hbm.at[pl.ds((hh * B + b) * HISTW, HISTW)], semo)
  out.start()
  out.wait()


def _sc_histograms(z):
  mesh = plsc.VectorSubcoreMesh(core_axis_name="c", subcore_axis_name="s")
  kern = pl.kernel(
      _sc_hist_body,
      out_type=jax.ShapeDtypeStruct((2 * FRAMES * NBINS,), jnp.float32),
      mesh=mesh,
      scratch_types=[
          pltpu.VMEM((3, 16, T), jnp.int32),
          pltpu.VMEM((3, 16, T), jnp.int32),
          pltpu.VMEM((HISTW,), jnp.float32),
          pltpu.SemaphoreType.DMA,
          pltpu.SemaphoreType.DMA,
          pltpu.SemaphoreType.DMA,
      ],
      compiler_params=pltpu.CompilerParams(needs_layout_passes=False),
  )
  return kern(z)


def _tc_body(hist_ref, wp_ref, bv_ref, out_ref):
  h = hist_ref[0, 0] + hist_ref[1, 0]
  sq = jnp.sum(h * h, axis=1, keepdims=True)
  inv = lax.rsqrt(jnp.maximum(sq, 1e-24))
  x = h * inv
  g = lax.dot_general(x, x, (((1,), (1,)), ((), ())),
                      preferred_element_type=jnp.float32)
  gp = jnp.concatenate(
      [jnp.zeros((T, 50), jnp.float32), g, jnp.zeros((T, 70), jnp.float32)],
      axis=1)
  row8 = lax.broadcasted_iota(jnp.int32, (8, 128), 0)
  blocks = []
  for i in range(T // 8):
    a = gp[8 * i:8 * i + 8, 8 * i:8 * i + 128]
    for bit in (1, 2, 4):
      a = jnp.where((row8 & bit) > 0, jnp.roll(a, -bit, axis=1), a)
    blocks.append(a)
  sims = jnp.concatenate(blocks, axis=0)
  out = jnp.dot(sims, wp_ref[...], preferred_element_type=jnp.float32)
  out_ref[0] = jnp.maximum(out + bv_ref[...], 0.0)


def _tc_similarity(hist2, wp, bv):
  return pl.pallas_call(
      _tc_body,
      grid=(B,),
      in_specs=[
          pl.BlockSpec((2, 1, T, NBINS), lambda i: (0, i, 0, 0)),
          pl.BlockSpec((128, ODIM), lambda i: (0, 0)),
          pl.BlockSpec((1, ODIM), lambda i: (0, 0)),
      ],
      out_specs=pl.BlockSpec((1, T, ODIM), lambda i: (i, 0, 0)),
      out_shape=jax.ShapeDtypeStruct((B, T, ODIM), jnp.float32),
  )(hist2, wp, bv)


@jax.jit
def kernel(inputs, W, b):
  z = jnp.transpose(inputs, (0, 2, 4, 3, 1))
  hist2 = _sc_histograms(z).reshape(2, B, T, NBINS)
  wp = jnp.concatenate(
      [W.T.astype(jnp.float32), jnp.zeros((128 - LOOKUP, ODIM), jnp.float32)],
      axis=0)
  bv = b.reshape(1, ODIM).astype(jnp.float32)
  return _tc_similarity(hist2, wp, bv)
